# Initial kernel scaffold; baseline (speedup 1.0000x reference)
import jax
import jax.numpy as jnp
from jax.experimental import pallas as pl


def kernel(x, y):
    raise NotImplementedError("write your pallas kernel here")


# speedup vs baseline: 18.6952x; 18.6952x over previous
import functools

import jax
import jax.numpy as jnp
from jax import lax
from jax.experimental import pallas as pl
from jax.experimental.pallas import tpu as pltpu, tpu_sc as plsc

_SQ3 = 0.5773502691896258

_NC, _NS, _L = 2, 16, 16
_NW = _NC * _NS


@functools.lru_cache(maxsize=None)
def _tp_kernel(B, C):
    XW = 4 * C
    OW = 8 * C
    b_per_w = B // _NW
    E = 50
    assert b_per_w % E == 0 and B % _NW == 0
    n_chunks = b_per_w // E
    G = C // _L

    mesh = plsc.VectorSubcoreMesh(
        core_axis_name="c", subcore_axis_name="s",
        num_cores=_NC, num_subcores=_NS)

    @functools.partial(
        pl.kernel,
        out_type=jax.ShapeDtypeStruct((B * OW,), jnp.float32),
        mesh=mesh,
        scratch_types=[
            pltpu.VMEM((E * XW,), jnp.float32),
            pltpu.VMEM((E * XW,), jnp.float32),
            pltpu.VMEM((E * OW,), jnp.float32),
        ],
    )
    def k(x_hbm, y_hbm, o_hbm, xv, yv, ov):
        wid = lax.axis_index("s") * _NC + lax.axis_index("c")
        base_edge = wid * b_per_w

        def chunk_body(ci, carry):
            e0 = base_edge + ci * E
            pltpu.sync_copy(x_hbm.at[pl.ds(e0 * XW, E * XW)], xv)
            pltpu.sync_copy(y_hbm.at[pl.ds(e0 * XW, E * XW)], yv)

            def edge_body(e, c2):
                xb = e * XW
                ob = e * OW
                for g in range(G):
                    c0 = g * _L
                    x0 = xv[pl.ds(xb + 0 * C + c0, _L)]
                    x1 = xv[pl.ds(xb + 1 * C + c0, _L)]
                    x2 = xv[pl.ds(xb + 2 * C + c0, _L)]
                    x3 = xv[pl.ds(xb + 3 * C + c0, _L)]
                    y0 = yv[pl.ds(xb + 0 * C + c0, _L)]
                    y1 = yv[pl.ds(xb + 1 * C + c0, _L)]
                    y2 = yv[pl.ds(xb + 2 * C + c0, _L)]
                    y3 = yv[pl.ds(xb + 3 * C + c0, _L)]
                    ov[pl.ds(ob + 0 * C + c0, _L)] = x0 * y0
                    ov[pl.ds(ob + 1 * C + c0, _L)] = x0 * y1
                    ov[pl.ds(ob + 2 * C + c0, _L)] = x0 * y2
                    ov[pl.ds(ob + 3 * C + c0, _L)] = x0 * y3
                    ov[pl.ds(ob + 4 * C + c0, _L)] = x1 * y0
                    ov[pl.ds(ob + 5 * C + c0, _L)] = x2 * y0
                    ov[pl.ds(ob + 6 * C + c0, _L)] = x3 * y0
                    ov[pl.ds(ob + 7 * C + c0, _L)] = (
                        x1 * y1 + x2 * y2 + x3 * y3) * _SQ3
                return c2

            lax.fori_loop(0, E, edge_body, 0)
            pltpu.sync_copy(ov, o_hbm.at[pl.ds(e0 * OW, E * OW)])
            return carry

        lax.fori_loop(0, n_chunks, chunk_body, 0)

    return k


def kernel(x, y):
    B, _, C = x.shape
    of = _tp_kernel(B, C)(x.reshape(-1), y.reshape(-1))
    return of.reshape(B, 8, C)


# speedup vs baseline: 40.2109x; 2.1509x over previous
import functools

import jax
import jax.numpy as jnp
from jax import lax
from jax.experimental import pallas as pl
from jax.experimental.pallas import tpu as pltpu, tpu_sc as plsc

_SQ3 = 0.5773502691896258

_NC, _NS, _L = 2, 16, 16
_NW = _NC * _NS


@functools.lru_cache(maxsize=None)
def _tp_kernel(B, C):
    XW = 4 * C
    OW = 8 * C
    b_per_w = B // _NW
    E = 25
    assert B % _NW == 0 and b_per_w % (2 * E) == 0
    n_chunks = b_per_w // E
    n_pairs = n_chunks // 2
    G = C // _L

    mesh = plsc.VectorSubcoreMesh(
        core_axis_name="c", subcore_axis_name="s",
        num_cores=_NC, num_subcores=_NS)

    @functools.partial(
        pl.kernel,
        out_type=jax.ShapeDtypeStruct((B * OW,), jnp.float32),
        mesh=mesh,
        scratch_types=[
            pltpu.VMEM((E * XW,), jnp.float32),
            pltpu.VMEM((E * XW,), jnp.float32),
            pltpu.VMEM((E * XW,), jnp.float32),
            pltpu.VMEM((E * XW,), jnp.float32),
            pltpu.VMEM((E * OW,), jnp.float32),
            pltpu.VMEM((E * OW,), jnp.float32),
            pltpu.SemaphoreType.DMA,
            pltpu.SemaphoreType.DMA,
            pltpu.SemaphoreType.DMA,
            pltpu.SemaphoreType.DMA,
            pltpu.SemaphoreType.DMA,
            pltpu.SemaphoreType.DMA,
        ],
    )
    def k(x_hbm, y_hbm, o_hbm,
          xv0, xv1, yv0, yv1, ov0, ov1, sx0, sx1, sy0, sy1, so0, so1):
        wid = lax.axis_index("s") * _NC + lax.axis_index("c")
        base = wid * b_per_w
        xvs, yvs, ovs = (xv0, xv1), (yv0, yv1), (ov0, ov1)
        sxs, sys, sos = (sx0, sx1), (sy0, sy1), (so0, so1)

        def start_in(b, ci):
            e0 = (base + ci * E) * XW
            pltpu.make_async_copy(
                x_hbm.at[pl.ds(e0, E * XW)], xvs[b], sxs[b]).start()
            pltpu.make_async_copy(
                y_hbm.at[pl.ds(e0, E * XW)], yvs[b], sys[b]).start()

        def wait_in(b):
            pltpu.make_async_copy(
                x_hbm.at[pl.ds(0, E * XW)], xvs[b], sxs[b]).wait()
            pltpu.make_async_copy(
                y_hbm.at[pl.ds(0, E * XW)], yvs[b], sys[b]).wait()

        def start_out(b, ci):
            e0 = (base + ci * E) * OW
            pltpu.make_async_copy(
                ovs[b], o_hbm.at[pl.ds(e0, E * OW)], sos[b]).start()

        def wait_out(b):
            pltpu.make_async_copy(
                ovs[b], o_hbm.at[pl.ds(0, E * OW)], sos[b]).wait()

        def compute(b):
            xv, yv, ov = xvs[b], yvs[b], ovs[b]

            def edge_body(e, c2):
                xb = e * XW
                ob = e * OW
                for g in range(G):
                    c0 = g * _L
                    x0 = xv[pl.ds(xb + 0 * C + c0, _L)]
                    x1 = xv[pl.ds(xb + 1 * C + c0, _L)]
                    x2 = xv[pl.ds(xb + 2 * C + c0, _L)]
                    x3 = xv[pl.ds(xb + 3 * C + c0, _L)]
                    y0 = yv[pl.ds(xb + 0 * C + c0, _L)]
                    y1 = yv[pl.ds(xb + 1 * C + c0, _L)]
                    y2 = yv[pl.ds(xb + 2 * C + c0, _L)]
                    y3 = yv[pl.ds(xb + 3 * C + c0, _L)]
                    ov[pl.ds(ob + 0 * C + c0, _L)] = x0 * y0
                    ov[pl.ds(ob + 1 * C + c0, _L)] = x0 * y1
                    ov[pl.ds(ob + 2 * C + c0, _L)] = x0 * y2
                    ov[pl.ds(ob + 3 * C + c0, _L)] = x0 * y3
                    ov[pl.ds(ob + 4 * C + c0, _L)] = x1 * y0
                    ov[pl.ds(ob + 5 * C + c0, _L)] = x2 * y0
                    ov[pl.ds(ob + 6 * C + c0, _L)] = x3 * y0
                    ov[pl.ds(ob + 7 * C + c0, _L)] = (
                        x1 * y1 + x2 * y2 + x3 * y3) * _SQ3
                return c2

            lax.fori_loop(0, E, edge_body, 0)

        start_in(0, 0)
        start_in(1, 1)

        for b in range(2):
            wait_in(b)
            compute(b)
            start_out(b, b)
            start_in(b, b + 2)

        def pair_body(pi, carry):
            for b in range(2):
                ci = pi * 2 + b
                wait_in(b)
                wait_out(b)
                compute(b)
                start_out(b, ci)
                start_in(b, ci + 2)
            return carry

        lax.fori_loop(1, n_pairs - 1, pair_body, 0)

        for b in range(2):
            wait_in(b)
            wait_out(b)
            compute(b)
            start_out(b, (n_pairs - 1) * 2 + b)
        for b in range(2):
            wait_out(b)

    return k


def kernel(x, y):
    B, _, C = x.shape
    of = _tp_kernel(B, C)(x.reshape(-1), y.reshape(-1))
    return of.reshape(B, 8, C)


# speedup vs baseline: 40.2751x; 1.0016x over previous
import functools

import jax
import jax.numpy as jnp
from jax import lax
from jax.experimental import pallas as pl
from jax.experimental.pallas import tpu as pltpu, tpu_sc as plsc

_SQ3 = 0.5773502691896258

_NC, _NS, _L = 2, 16, 16
_NW = _NC * _NS


@functools.lru_cache(maxsize=None)
def _tp_kernel(B, C):
    XW = 4 * C
    OW = 8 * C
    b_per_w = B // _NW
    E = 25
    assert B % _NW == 0 and b_per_w % (2 * E) == 0
    n_chunks = b_per_w // E
    n_pairs = n_chunks // 2
    G = C // _L

    mesh = plsc.VectorSubcoreMesh(
        core_axis_name="c", subcore_axis_name="s",
        num_cores=_NC, num_subcores=_NS)

    @functools.partial(
        pl.kernel,
        out_type=jax.ShapeDtypeStruct((B * OW,), jnp.float32),
        mesh=mesh,
        scratch_types=[
            pltpu.VMEM((E * XW,), jnp.float32),
            pltpu.VMEM((E * XW,), jnp.float32),
            pltpu.VMEM((E * XW,), jnp.float32),
            pltpu.VMEM((E * XW,), jnp.float32),
            pltpu.VMEM((E * OW,), jnp.float32),
            pltpu.VMEM((E * OW,), jnp.float32),
            pltpu.SemaphoreType.DMA,
            pltpu.SemaphoreType.DMA,
            pltpu.SemaphoreType.DMA,
            pltpu.SemaphoreType.DMA,
            pltpu.SemaphoreType.DMA,
            pltpu.SemaphoreType.DMA,
        ],
    )
    def k(x_hbm, y_hbm, o_hbm,
          xv0, xv1, yv0, yv1, ov0, ov1, sx0, sx1, sy0, sy1, so0, so1):
        wid = lax.axis_index("s") * _NC + lax.axis_index("c")
        base = wid * b_per_w
        xvs, yvs, ovs = (xv0, xv1), (yv0, yv1), (ov0, ov1)
        sxs, sys, sos = (sx0, sx1), (sy0, sy1), (so0, so1)

        def start_in(b, ci):
            e0 = (base + ci * E) * XW
            pltpu.make_async_copy(
                x_hbm.at[pl.ds(e0, E * XW)], xvs[b], sxs[b]).start()
            pltpu.make_async_copy(
                y_hbm.at[pl.ds(e0, E * XW)], yvs[b], sys[b]).start()

        def wait_in(b):
            pltpu.make_async_copy(
                x_hbm.at[pl.ds(0, E * XW)], xvs[b], sxs[b]).wait()
            pltpu.make_async_copy(
                y_hbm.at[pl.ds(0, E * XW)], yvs[b], sys[b]).wait()

        def start_out(b, ci):
            e0 = (base + ci * E) * OW
            pltpu.make_async_copy(
                ovs[b], o_hbm.at[pl.ds(e0, E * OW)], sos[b]).start()

        def wait_out(b):
            pltpu.make_async_copy(
                ovs[b], o_hbm.at[pl.ds(0, E * OW)], sos[b]).wait()

        def compute(b):
            xv, yv, ov = xvs[b], yvs[b], ovs[b]

            @plsc.parallel_loop(0, E, step=1, unroll=2)
            def edge_body(e):
                xb = e * XW
                ob = e * OW
                for g in range(G):
                    c0 = g * _L
                    x0 = xv[pl.ds(xb + 0 * C + c0, _L)]
                    x1 = xv[pl.ds(xb + 1 * C + c0, _L)]
                    x2 = xv[pl.ds(xb + 2 * C + c0, _L)]
                    x3 = xv[pl.ds(xb + 3 * C + c0, _L)]
                    y0 = yv[pl.ds(xb + 0 * C + c0, _L)]
                    y1 = yv[pl.ds(xb + 1 * C + c0, _L)]
                    y2 = yv[pl.ds(xb + 2 * C + c0, _L)]
                    y3 = yv[pl.ds(xb + 3 * C + c0, _L)]
                    ov[pl.ds(ob + 0 * C + c0, _L)] = x0 * y0
                    ov[pl.ds(ob + 1 * C + c0, _L)] = x0 * y1
                    ov[pl.ds(ob + 2 * C + c0, _L)] = x0 * y2
                    ov[pl.ds(ob + 3 * C + c0, _L)] = x0 * y3
                    ov[pl.ds(ob + 4 * C + c0, _L)] = x1 * y0
                    ov[pl.ds(ob + 5 * C + c0, _L)] = x2 * y0
                    ov[pl.ds(ob + 6 * C + c0, _L)] = x3 * y0
                    ov[pl.ds(ob + 7 * C + c0, _L)] = (
                        x1 * y1 + x2 * y2 + x3 * y3) * _SQ3

        start_in(0, 0)
        start_in(1, 1)

        for b in range(2):
            wait_in(b)
            compute(b)
            start_out(b, b)
            start_in(b, b + 2)

        def pair_body(pi, carry):
            for b in range(2):
                ci = pi * 2 + b
                wait_in(b)
                wait_out(b)
                compute(b)
                start_out(b, ci)
                start_in(b, ci + 2)
            return carry

        lax.fori_loop(1, n_pairs - 1, pair_body, 0)

        for b in range(2):
            wait_in(b)
            wait_out(b)
            compute(b)
            start_out(b, (n_pairs - 1) * 2 + b)
        for b in range(2):
            wait_out(b)

    return k


def kernel(x, y):
    B, _, C = x.shape
    of = _tp_kernel(B, C)(x.reshape(-1), y.reshape(-1))
    return of.reshape(B, 8, C)
